# baseline (speedup 1.0000x reference)
import functools

import jax
import jax.numpy as jnp
import numpy as np
from jax import lax
from jax.experimental import pallas as pl
from jax.experimental.pallas import tpu as pltpu
from jax.experimental.pallas import tpu_sc as plsc

B, S = 4096, 200
D = 64
N_TOK = B * S
NW = 32
TPW = N_TOK // NW
C = 128
NCH = TPW // C
T1_ROWS = 60 * 24
T2_ROWS = 31 * 7 * 12
T2_PAD = 2608


def _onehot(idx, n):
    return (idx[:, None] == np.arange(n)[None, :]).astype(np.float32)


_r1 = np.arange(T1_ROWS)
_G1M = _onehot(_r1 // 24, 60)
_G1H = _onehot(_r1 % 24, 24)
_r2 = np.arange(T2_PAD)
_G2D = _onehot(np.minimum(_r2 // 84, 30), 31)
_G2W = _onehot((_r2 // 12) % 7, 7)
_G2MO = _onehot(_r2 % 12, 12)


def _tables_body(em, eh, ed, ew, emo, w, bias, g1m, g1h, g2d, g2w, g2mo,
                 t1_ref, t2_ref):
    f32 = jnp.float32
    am = jnp.dot(em[...], w[0:12, :], preferred_element_type=f32)
    ah = jnp.dot(eh[...], w[12:24, :], preferred_element_type=f32)
    ad = jnp.dot(ed[...], w[24:36, :], preferred_element_type=f32)
    aw = jnp.dot(ew[...], w[36:48, :], preferred_element_type=f32)
    amo = jnp.dot(emo[...], w[48:60, :], preferred_element_type=f32)
    t1_ref[:, 0:D] = (jnp.dot(g1m[...], am, preferred_element_type=f32)
                      + jnp.dot(g1h[...], ah, preferred_element_type=f32)
                      + bias[...])
    t1_ref[:, D:2 * D] = jnp.zeros((T1_ROWS, D), f32)
    t2_ref[:, 0:D] = (jnp.dot(g2d[...], ad, preferred_element_type=f32)
                      + jnp.dot(g2w[...], aw, preferred_element_type=f32)
                      + jnp.dot(g2mo[...], amo, preferred_element_type=f32))
    t2_ref[:, D:2 * D] = jnp.zeros((T2_PAD, D), f32)


def _build_tables(E_minute, E_hour, E_day, E_week, E_month, W, b):
    return pl.pallas_call(
        _tables_body,
        out_shape=(
            jax.ShapeDtypeStruct((T1_ROWS, 2 * D), jnp.float32),
            jax.ShapeDtypeStruct((T2_PAD, 2 * D), jnp.float32),
        ),
    )(E_minute, E_hour, E_day, E_week, E_month, W, b.reshape(1, D),
      _G1M, _G1H, _G2D, _G2W, _G2MO)


def _sc_body(m_hbm, h_hbm, d_hbm, w_hbm, mo_hbm, t1_hbm, t2_hbm, out_hbm,
             rawv, i1v, i2v, g1v, r2v, packv,
             si0, si1, sg0, sg1, sh0, sh1, sw0, sw1):
    wid = lax.axis_index("s") * 2 + lax.axis_index("c")
    wbase = wid * TPW
    wbase2 = wbase // 2
    idx_hbms = (m_hbm, h_hbm, d_hbm, w_hbm, mo_hbm)
    sidx = (si0, si1)
    sg_t1 = (sg0, sg1)
    sg_t2 = (sh0, sh1)
    swb = (sw0, sw1)

    def issue_idx(g, slot):
        base = pl.multiple_of(wbase + g * C, C)
        for k in range(5):
            pltpu.make_async_copy(idx_hbms[k].at[pl.ds(base, C)],
                                  rawv.at[slot, k], sidx[slot]).start()

    def wait_idx(slot):
        for k in range(5):
            pltpu.make_async_copy(idx_hbms[k].at[pl.ds(0, C)],
                                  rawv.at[slot, k], sidx[slot]).wait()

    def combine(slot):
        for g in range(C // 16):
            sl = pl.ds(g * 16, 16)
            i1v[slot, sl] = rawv[slot, 0, sl] * 24 + rawv[slot, 1, sl]
            i2v[slot, sl] = (rawv[slot, 2, sl] * 84 + rawv[slot, 3, sl] * 12
                             + rawv[slot, 4, sl])

    def issue_gathers(slot):
        pltpu.make_async_copy(t1_hbm.at[i1v.at[slot]], g1v.at[slot],
                              sg_t1[slot]).start()
        pltpu.make_async_copy(t2_hbm.at[i2v.at[slot]], r2v.at[slot],
                              sg_t2[slot]).start()

    def wait_gathers(slot):
        pltpu.make_async_copy(t1_hbm.at[pl.ds(0, C)], g1v.at[slot],
                              sg_t1[slot]).wait()
        pltpu.make_async_copy(t2_hbm.at[pl.ds(0, C)], r2v.at[slot],
                              sg_t2[slot]).wait()

    def accumulate(slot):
        def acc(p, _):
            for half in range(2):
                for q in range(D // 16):
                    src = pl.ds(q * 16, 16)
                    dst = pl.ds(half * D + q * 16, 16)
                    t = p * 2 + half
                    packv[slot, p, dst] = (g1v[slot, t, src]
                                           + r2v[slot, t, src])
            return 0
        lax.fori_loop(0, C // 2, acc, 0, unroll=2)

    def issue_wb(g, slot):
        base2 = pl.multiple_of(wbase2 + g * (C // 2), C // 2)
        pltpu.make_async_copy(packv.at[slot], out_hbm.at[pl.ds(base2, C // 2)],
                              swb[slot]).start()

    def wait_wb(slot):
        pltpu.make_async_copy(packv.at[slot], out_hbm.at[pl.ds(0, C // 2)],
                              swb[slot]).wait()

    def body(g, buf, first=False, last_pair=False, no_idx=False):
        nbuf = 1 - buf
        if not last_pair:
            wait_idx(nbuf)
            combine(nbuf)
            issue_gathers(nbuf)
            if not no_idx:
                issue_idx(g + 2, buf)
        wait_gathers(buf)
        if not first:
            wait_wb(buf)
        accumulate(buf)
        issue_wb(g, buf)

    issue_idx(0, 0)
    wait_idx(0)
    combine(0)
    issue_gathers(0)
    issue_idx(1, 1)
    body(0, 0, first=True)
    body(1, 1, first=True)

    def pair(p, _):
        g = 2 * p + 2
        body(g, 0)
        body(g + 1, 1)
        return 0

    lax.fori_loop(0, (NCH - 4) // 2, pair, 0)
    body(NCH - 2, 0, no_idx=True)
    body(NCH - 1, 1, last_pair=True)
    wait_wb(0)
    wait_wb(1)


@jax.jit
def _run(minute, hour, day, week, month,
         E_minute, E_hour, E_day, E_week, E_month, W, b):
    t1, t2 = _build_tables(E_minute, E_hour, E_day, E_week, E_month, W, b)
    mesh = plsc.VectorSubcoreMesh(core_axis_name="c", subcore_axis_name="s")
    sc = pl.kernel(
        _sc_body,
        out_type=jax.ShapeDtypeStruct((N_TOK // 2, 2 * D), jnp.float32),
        mesh=mesh,
        scratch_types=[
            pltpu.VMEM((2, 5, C), jnp.int32),
            pltpu.VMEM((2, C), jnp.int32),
            pltpu.VMEM((2, C), jnp.int32),
            pltpu.VMEM((2, C, 2 * D), jnp.float32),
            pltpu.VMEM((2, C, 2 * D), jnp.float32),
            pltpu.VMEM((2, C // 2, 2 * D), jnp.float32),
        ] + [pltpu.SemaphoreType.DMA] * 8,
    )
    out = sc(minute.reshape(-1), hour.reshape(-1), day.reshape(-1),
             week.reshape(-1), month.reshape(-1), t1, t2)
    return out.reshape(B, S, D)


def kernel(minute, hour, day, week, month,
           E_minute, E_hour, E_day, E_week, E_month, W, b):
    return _run(minute, hour, day, week, month,
                E_minute, E_hour, E_day, E_week, E_month, W, b)


# speedup vs baseline: 1.3392x; 1.3392x over previous
import functools

import jax
import jax.numpy as jnp
import numpy as np
from jax import lax
from jax.experimental import pallas as pl
from jax.experimental.pallas import tpu as pltpu
from jax.experimental.pallas import tpu_sc as plsc

B, S = 4096, 200
D = 64
N_TOK = B * S
NW = 32
TPW = N_TOK // NW
C = 128
NCH = TPW // C
T1_ROWS = 60 * 24
T2_ROWS = 31 * 7 * 12
T2_PAD = 2608


def _onehot(idx, n):
    return (idx[:, None] == np.arange(n)[None, :]).astype(np.float32)


_r1 = np.arange(T1_ROWS)
_G1M = _onehot(_r1 // 24, 60)
_G1H = _onehot(_r1 % 24, 24)
_r2 = np.arange(T2_PAD)
_G2D = _onehot(np.minimum(_r2 // 84, 30), 31)
_G2W = _onehot((_r2 // 12) % 7, 7)
_G2MO = _onehot(_r2 % 12, 12)


def _tables_body(em, eh, ed, ew, emo, w, bias, g1m, g1h, g2d, g2w, g2mo,
                 t1_ref, t2_ref):
    f32 = jnp.float32
    am = jnp.dot(em[...], w[0:12, :], preferred_element_type=f32)
    ah = jnp.dot(eh[...], w[12:24, :], preferred_element_type=f32)
    ad = jnp.dot(ed[...], w[24:36, :], preferred_element_type=f32)
    aw = jnp.dot(ew[...], w[36:48, :], preferred_element_type=f32)
    amo = jnp.dot(emo[...], w[48:60, :], preferred_element_type=f32)
    t1_ref[:, 0:D] = (jnp.dot(g1m[...], am, preferred_element_type=f32)
                      + jnp.dot(g1h[...], ah, preferred_element_type=f32)
                      + bias[...])
    t1_ref[:, D:2 * D] = jnp.zeros((T1_ROWS, D), f32)
    t2_ref[:, 0:D] = (jnp.dot(g2d[...], ad, preferred_element_type=f32)
                      + jnp.dot(g2w[...], aw, preferred_element_type=f32)
                      + jnp.dot(g2mo[...], amo, preferred_element_type=f32))
    t2_ref[:, D:2 * D] = jnp.zeros((T2_PAD, D), f32)


def _build_tables(E_minute, E_hour, E_day, E_week, E_month, W, b):
    return pl.pallas_call(
        _tables_body,
        out_shape=(
            jax.ShapeDtypeStruct((T1_ROWS, 2 * D), jnp.float32),
            jax.ShapeDtypeStruct((T2_PAD, 2 * D), jnp.float32),
        ),
    )(E_minute, E_hour, E_day, E_week, E_month, W, b.reshape(1, D),
      _G1M, _G1H, _G2D, _G2W, _G2MO)


def _sc_body(m_hbm, h_hbm, d_hbm, w_hbm, mo_hbm, t1_hbm, t2_hbm, out_hbm,
             rawv, i1v, i2v, g1v, r2v, packv,
             si0, si1, sg0, sg1, sh0, sh1, sw0, sw1):
    wid = lax.axis_index("s") * 2 + lax.axis_index("c")
    wbase = wid * TPW
    wbase2 = wbase // 2
    idx_hbms = (m_hbm, h_hbm, d_hbm, w_hbm, mo_hbm)
    sidx = (si0, si1)
    sg_t1 = (sg0, sg1)
    sg_t2 = (sh0, sh1)
    swb = (sw0, sw1)

    def issue_idx(g, slot):
        base = pl.multiple_of(wbase + g * C, C)
        for k in range(5):
            pltpu.make_async_copy(idx_hbms[k].at[pl.ds(base, C)],
                                  rawv.at[slot, k], sidx[slot]).start()

    def wait_idx(slot):
        for k in range(5):
            pltpu.make_async_copy(idx_hbms[k].at[pl.ds(0, C)],
                                  rawv.at[slot, k], sidx[slot]).wait()

    def combine(slot):
        for g in range(C // 16):
            sl = pl.ds(g * 16, 16)
            i1v[slot, sl] = rawv[slot, 0, sl] * 24 + rawv[slot, 1, sl]
            i2v[slot, sl] = (rawv[slot, 2, sl] * 84 + rawv[slot, 3, sl] * 12
                             + rawv[slot, 4, sl])

    def issue_gathers(slot):
        pltpu.make_async_copy(t1_hbm.at[i1v.at[slot]], g1v.at[slot],
                              sg_t1[slot]).start()
        pltpu.make_async_copy(t2_hbm.at[i2v.at[slot]], r2v.at[slot],
                              sg_t2[slot]).start()

    def wait_gathers(slot):
        pltpu.make_async_copy(t1_hbm.at[pl.ds(0, C)], g1v.at[slot],
                              sg_t1[slot]).wait()
        pltpu.make_async_copy(t2_hbm.at[pl.ds(0, C)], r2v.at[slot],
                              sg_t2[slot]).wait()

    def accumulate(slot):
        def acc(t8, _):
            for tt in range(4):
                for q in range(D // 16):
                    sl = pl.ds(q * 16, 16)
                    t = t8 * 4 + tt
                    packv[slot, t, sl] = (g1v[slot, t, sl]
                                          + r2v[slot, t, sl])
            return 0
        lax.fori_loop(0, C // 4, acc, 0, unroll=2)

    def issue_wb(g, slot):
        base = pl.multiple_of(wbase + g * C, C)
        pltpu.make_async_copy(packv.at[slot], out_hbm.at[pl.ds(base, C)],
                              swb[slot]).start()

    def wait_wb(slot):
        pltpu.make_async_copy(packv.at[slot], out_hbm.at[pl.ds(0, C)],
                              swb[slot]).wait()

    def body(g, buf, first=False, last_pair=False, no_idx=False):
        nbuf = 1 - buf
        if not last_pair:
            wait_idx(nbuf)
            combine(nbuf)
            issue_gathers(nbuf)
            if not no_idx:
                issue_idx(g + 2, buf)
        wait_gathers(buf)
        if not first:
            wait_wb(buf)
        accumulate(buf)
        issue_wb(g, buf)

    issue_idx(0, 0)
    wait_idx(0)
    combine(0)
    issue_gathers(0)
    issue_idx(1, 1)
    body(0, 0, first=True)
    body(1, 1, first=True)

    def pair(p, _):
        g = 2 * p + 2
        body(g, 0)
        body(g + 1, 1)
        return 0

    lax.fori_loop(0, (NCH - 4) // 2, pair, 0)
    body(NCH - 2, 0, no_idx=True)
    body(NCH - 1, 1, last_pair=True)
    wait_wb(0)
    wait_wb(1)


@jax.jit
def _run(minute, hour, day, week, month,
         E_minute, E_hour, E_day, E_week, E_month, W, b):
    t1, t2 = _build_tables(E_minute, E_hour, E_day, E_week, E_month, W, b)
    mesh = plsc.VectorSubcoreMesh(core_axis_name="c", subcore_axis_name="s")
    sc = pl.kernel(
        _sc_body,
        out_type=jax.ShapeDtypeStruct((N_TOK, D), jnp.float32),
        mesh=mesh,
        scratch_types=[
            pltpu.VMEM((2, 5, C), jnp.int32),
            pltpu.VMEM((2, C), jnp.int32),
            pltpu.VMEM((2, C), jnp.int32),
            pltpu.VMEM((2, C, 2 * D), jnp.float32),
            pltpu.VMEM((2, C, 2 * D), jnp.float32),
            pltpu.VMEM((2, C, D), jnp.float32),
        ] + [pltpu.SemaphoreType.DMA] * 8,
    )
    out = sc(minute.reshape(-1), hour.reshape(-1), day.reshape(-1),
             week.reshape(-1), month.reshape(-1), t1, t2)
    return out.reshape(B, S, D)


def kernel(minute, hour, day, week, month,
           E_minute, E_hour, E_day, E_week, E_month, W, b):
    return _run(minute, hour, day, week, month,
                E_minute, E_hour, E_day, E_week, E_month, W, b)


# speedup vs baseline: 1.5128x; 1.1296x over previous
import functools

import jax
import jax.numpy as jnp
import numpy as np
from jax import lax
from jax.experimental import pallas as pl
from jax.experimental.pallas import tpu as pltpu
from jax.experimental.pallas import tpu_sc as plsc

B, S = 4096, 200
D = 64
N_TOK = B * S
NW = 32
TPW = N_TOK // NW
C = 128
NCH = TPW // C
T1_ROWS = 60 * 24
T2_ROWS = 31 * 7 * 12
T2_PAD = 2608


def _onehot(idx, n):
    return (idx[:, None] == np.arange(n)[None, :]).astype(np.float32)


_r1 = np.arange(T1_ROWS)
_G1M = _onehot(_r1 // 24, 60)
_G1H = _onehot(_r1 % 24, 24)
_r2 = np.arange(T2_PAD)
_G2D = _onehot(np.minimum(_r2 // 84, 30), 31)
_G2W = _onehot((_r2 // 12) % 7, 7)
_G2MO = _onehot(_r2 % 12, 12)


def _tables_body(em, eh, ed, ew, emo, w, bias, g1m, g1h, g2d, g2w, g2mo,
                 t1_ref, t2_ref):
    f32 = jnp.float32
    am = jnp.dot(em[...], w[0:12, :], preferred_element_type=f32)
    ah = jnp.dot(eh[...], w[12:24, :], preferred_element_type=f32)
    ad = jnp.dot(ed[...], w[24:36, :], preferred_element_type=f32)
    aw = jnp.dot(ew[...], w[36:48, :], preferred_element_type=f32)
    amo = jnp.dot(emo[...], w[48:60, :], preferred_element_type=f32)
    t1_ref[:, 0:D] = (jnp.dot(g1m[...], am, preferred_element_type=f32)
                      + jnp.dot(g1h[...], ah, preferred_element_type=f32)
                      + bias[...])
    t1_ref[:, D:2 * D] = jnp.zeros((T1_ROWS, D), f32)
    t2_ref[:, 0:D] = (jnp.dot(g2d[...], ad, preferred_element_type=f32)
                      + jnp.dot(g2w[...], aw, preferred_element_type=f32)
                      + jnp.dot(g2mo[...], amo, preferred_element_type=f32))
    t2_ref[:, D:2 * D] = jnp.zeros((T2_PAD, D), f32)


def _build_tables(E_minute, E_hour, E_day, E_week, E_month, W, b):
    return pl.pallas_call(
        _tables_body,
        out_shape=(
            jax.ShapeDtypeStruct((T1_ROWS, 2 * D), jnp.float32),
            jax.ShapeDtypeStruct((T2_PAD, 2 * D), jnp.float32),
        ),
    )(E_minute, E_hour, E_day, E_week, E_month, W, b.reshape(1, D),
      _G1M, _G1H, _G2D, _G2W, _G2MO)


def _sc_body(m_hbm, h_hbm, d_hbm, w_hbm, mo_hbm, t1_hbm, t2_hbm, out_hbm,
             rawv, i1v, i2v, g1v, r2v, packv, tabv,
             si0, si1, sg0, sg1, sh0, sh1, sw0, sw1):
    wid = lax.axis_index("s") * 2 + lax.axis_index("c")
    wbase = wid * TPW
    idx_hbms = (m_hbm, h_hbm, d_hbm, w_hbm, mo_hbm)
    sidx = (si0, si1)
    sg_t1 = (sg0, sg1)
    sg_t2 = (sh0, sh1)
    swb = (sw0, sw1)

    @pl.when(lax.axis_index("s") == 0)
    def _stage():
        pltpu.sync_copy(t2_hbm, tabv)
    plsc.subcore_barrier()

    def issue_idx(g, slot):
        base = pl.multiple_of(wbase + g * C, C)
        for k in range(5):
            pltpu.make_async_copy(idx_hbms[k].at[pl.ds(base, C)],
                                  rawv.at[slot, k], sidx[slot]).start()

    def wait_idx(slot):
        for k in range(5):
            pltpu.make_async_copy(idx_hbms[k].at[pl.ds(0, C)],
                                  rawv.at[slot, k], sidx[slot]).wait()

    def combine(slot):
        for g in range(C // 16):
            sl = pl.ds(g * 16, 16)
            i1v[slot, sl] = rawv[slot, 0, sl] * 24 + rawv[slot, 1, sl]
            i2v[slot, sl] = (rawv[slot, 2, sl] * 84 + rawv[slot, 3, sl] * 12
                             + rawv[slot, 4, sl])

    def issue_gathers(slot):
        pltpu.make_async_copy(t1_hbm.at[i1v.at[slot]], g1v.at[slot],
                              sg_t1[slot]).start()
        pltpu.make_async_copy(tabv.at[i2v.at[slot]], r2v.at[slot],
                              sg_t2[slot]).start()

    def wait_gathers(slot):
        pltpu.make_async_copy(t1_hbm.at[pl.ds(0, C)], g1v.at[slot],
                              sg_t1[slot]).wait()
        pltpu.make_async_copy(tabv.at[pl.ds(0, C)], r2v.at[slot],
                              sg_t2[slot]).wait()

    def accumulate(slot):
        def acc(t8, _):
            for tt in range(4):
                for q in range(D // 16):
                    sl = pl.ds(q * 16, 16)
                    t = t8 * 4 + tt
                    packv[slot, t, sl] = (g1v[slot, t, sl]
                                          + r2v[slot, t, sl])
            return 0
        lax.fori_loop(0, C // 4, acc, 0, unroll=2)

    def issue_wb(g, slot):
        base = pl.multiple_of(wbase + g * C, C)
        pltpu.make_async_copy(packv.at[slot], out_hbm.at[pl.ds(base, C)],
                              swb[slot]).start()

    def wait_wb(slot):
        pltpu.make_async_copy(packv.at[slot], out_hbm.at[pl.ds(0, C)],
                              swb[slot]).wait()

    def body(g, buf, first=False, last_pair=False, no_idx=False):
        nbuf = 1 - buf
        if not last_pair:
            wait_idx(nbuf)
            combine(nbuf)
            issue_gathers(nbuf)
            if not no_idx:
                issue_idx(g + 2, buf)
        wait_gathers(buf)
        if not first:
            wait_wb(buf)
        accumulate(buf)
        issue_wb(g, buf)

    issue_idx(0, 0)
    wait_idx(0)
    combine(0)
    issue_gathers(0)
    issue_idx(1, 1)
    body(0, 0, first=True)
    body(1, 1, first=True)

    def pair(p, _):
        g = 2 * p + 2
        body(g, 0)
        body(g + 1, 1)
        return 0

    lax.fori_loop(0, (NCH - 4) // 2, pair, 0)
    body(NCH - 2, 0, no_idx=True)
    body(NCH - 1, 1, last_pair=True)
    wait_wb(0)
    wait_wb(1)


@jax.jit
def _run(minute, hour, day, week, month,
         E_minute, E_hour, E_day, E_week, E_month, W, b):
    t1, t2 = _build_tables(E_minute, E_hour, E_day, E_week, E_month, W, b)
    mesh = plsc.VectorSubcoreMesh(core_axis_name="c", subcore_axis_name="s")
    sc = pl.kernel(
        _sc_body,
        out_type=jax.ShapeDtypeStruct((N_TOK, D), jnp.float32),
        mesh=mesh,
        scratch_types=[
            pltpu.VMEM((2, 5, C), jnp.int32),
            pltpu.VMEM((2, C), jnp.int32),
            pltpu.VMEM((2, C), jnp.int32),
            pltpu.VMEM((2, C, 2 * D), jnp.float32),
            pltpu.VMEM((2, C, 2 * D), jnp.float32),
            pltpu.VMEM((2, C, D), jnp.float32),
            pltpu.VMEM_SHARED((T2_PAD, 2 * D), jnp.float32),
        ] + [pltpu.SemaphoreType.DMA] * 8,
    )
    out = sc(minute.reshape(-1), hour.reshape(-1), day.reshape(-1),
             week.reshape(-1), month.reshape(-1), t1, t2)
    return out.reshape(B, S, D)


def kernel(minute, hour, day, week, month,
           E_minute, E_hour, E_day, E_week, E_month, W, b):
    return _run(minute, hour, day, week, month,
                E_minute, E_hour, E_day, E_week, E_month, W, b)
